# baseline (speedup 1.0000x reference)
import jax
import jax.numpy as jnp
from jax.experimental import pallas as pl
from jax.experimental.pallas import tpu as pltpu

_TILE = 64


def _route_body(eid_ref, slot_ref, starts_ref, nblocks_ref):
    n = eid_ref.shape[0]
    num_e = starts_ref.shape[1]
    eid = eid_ref[...]
    lanes = jax.lax.broadcasted_iota(jnp.int32, (n, num_e), 1)
    oh = eid == lanes
    oh_bf = oh.astype(jnp.bfloat16)
    oh_f = oh.astype(jnp.float32)

    row = jax.lax.broadcasted_iota(jnp.int32, (n, n), 0)
    col = jax.lax.broadcasted_iota(jnp.int32, (n, n), 1)
    lower = (col < row).astype(jnp.bfloat16)
    before = jnp.dot(lower, oh_bf, preferred_element_type=jnp.float32)
    rank = jnp.sum(before * oh_f, axis=1, keepdims=True)

    counts = jnp.sum(oh_f, axis=0, keepdims=True).astype(jnp.int32)
    aligned = ((counts + 7) // 8) * 8
    erow = jax.lax.broadcasted_iota(jnp.int32, (num_e, num_e), 0)
    ecol = jax.lax.broadcasted_iota(jnp.int32, (num_e, num_e), 1)
    tri = (erow < ecol).astype(jnp.bfloat16)
    starts_f = jnp.dot(aligned.astype(jnp.bfloat16), tri,
                       preferred_element_type=jnp.float32)
    start_of_tok = jnp.sum(starts_f * oh_f, axis=1, keepdims=True)

    slot_ref[...] = (start_of_tok + rank).astype(jnp.int32)
    starts_ref[...] = starts_f.astype(jnp.int32)
    nblocks_ref[...] = (counts + (_TILE - 1)) // _TILE


def _moe_body(starts_ref, nblocks_ref, xs_ref, xw_ref, wg_ref, wu_ref,
              wd_ref, out_ref):
    e = pl.program_id(0)
    start = starts_ref[e]
    nb = nblocks_ref[e]
    wg = wg_ref[0].astype(jnp.bfloat16)
    wu = wu_ref[0].astype(jnp.bfloat16)
    wd = wd_ref[0].astype(jnp.bfloat16)

    def tile(k, carry):
        offs = pl.multiple_of(start + k * _TILE, 8)
        x = xs_ref[pl.ds(offs, _TILE), :].astype(jnp.bfloat16)
        xw = xw_ref[pl.ds(offs, _TILE), :].astype(jnp.bfloat16)
        g = jnp.dot(x, wg, preferred_element_type=jnp.float32)
        u = jnp.dot(xw, wu, preferred_element_type=jnp.float32)
        a = ((g * jax.nn.sigmoid(g)) * u).astype(jnp.bfloat16)
        out_ref[pl.ds(offs, _TILE), :] = jnp.dot(
            a, wd, preferred_element_type=jnp.float32)
        return carry

    jax.lax.fori_loop(0, nb, tile, 0)


def kernel(hidden_states, top_k_indices, top_k_weights, Wg, Wu, Wd):
    N, D = hidden_states.shape
    E, _, H = Wg.shape
    K = top_k_indices.shape[1]
    NK = N * K

    eid = top_k_indices.reshape(NK, 1).astype(jnp.int32)
    wts = top_k_weights.reshape(NK).astype(jnp.float32)

    slot, starts, nblocks = pl.pallas_call(
        _route_body,
        out_shape=(
            jax.ShapeDtypeStruct((NK, 1), jnp.int32),
            jax.ShapeDtypeStruct((1, E), jnp.int32),
            jax.ShapeDtypeStruct((1, E), jnp.int32),
        ),
    )(eid)
    slot = slot.reshape(NK)

    npad = NK + 8 * E + 4 * _TILE
    npad = ((npad + 255) // 256) * 256

    if K > 1:
        toks = jnp.repeat(jnp.arange(N, dtype=jnp.int32), K)
        hs = hidden_states[toks]
    else:
        hs = hidden_states
    hw = hs * wts[:, None]
    xs = jnp.zeros((npad, D), jnp.float32).at[slot].set(hs)
    xw = jnp.zeros((npad, D), jnp.float32).at[slot].set(hw)

    ys = pl.pallas_call(
        _moe_body,
        grid_spec=pltpu.PrefetchScalarGridSpec(
            num_scalar_prefetch=2,
            grid=(E,),
            in_specs=[
                pl.BlockSpec((npad, D), lambda e, s, nb: (0, 0)),
                pl.BlockSpec((npad, D), lambda e, s, nb: (0, 0)),
                pl.BlockSpec((1, D, H), lambda e, s, nb: (e, 0, 0)),
                pl.BlockSpec((1, D, H), lambda e, s, nb: (e, 0, 0)),
                pl.BlockSpec((1, H, D), lambda e, s, nb: (e, 0, 0)),
            ],
            out_specs=pl.BlockSpec((npad, D), lambda e, s, nb: (0, 0)),
        ),
        out_shape=jax.ShapeDtypeStruct((npad, D), jnp.float32),
        compiler_params=pltpu.CompilerParams(
            dimension_semantics=("arbitrary",)),
    )(starts.reshape(E), nblocks.reshape(E), xs, xw, Wg, Wu, Wd)

    slot2 = slot.reshape(N, K)
    out = ys[slot2[:, 0]]
    for k in range(1, K):
        out = out + ys[slot2[:, k]]
    return out
